# baseline (speedup 1.0000x reference)
import jax
import jax.numpy as jnp
from jax.experimental import pallas as pl
from jax.experimental.pallas import tpu as pltpu

_B = 16
_C = 3
_D = 384
_M = 512
_P = _M - _D
_N = _B * _C
_G = 8
_NG = _N // _G
_KG = 3
_K = _KG * _G


def _pad_kernel(d_hbm, o_hbm, buf, in_sems, out_sems):
    for k in range(_K):
        buf[k, : _D, _D :] = jnp.full((_D, _P), jnp.nan, jnp.float32)
        buf[k, _D :, :] = jnp.full((_P, _M), jnp.nan, jnp.float32)

    ins = [
        pltpu.make_async_copy(
            d_hbm.at[pl.ds(g * _G, _G)],
            buf.at[pl.ds((g % _KG) * _G, _G), pl.ds(0, _D), pl.ds(0, _D)],
            in_sems.at[g % _KG],
        )
        for g in range(_NG)
    ]
    outs = [
        pltpu.make_async_copy(
            buf.at[pl.ds((g % _KG) * _G, _G)],
            o_hbm.at[pl.ds(g * _G, _G)],
            out_sems.at[g % _KG],
        )
        for g in range(_NG)
    ]
    for g in range(_KG):
        ins[g].start(priority=g % 2)
    for g in range(_NG):
        ins[g].wait()
        outs[g].start(priority=g % 2)
        gd = g - 1
        if gd >= 0 and gd + _KG < _NG:
            outs[gd].wait()
            ins[gd + _KG].start(priority=(gd + _KG) % 2)
    for g in range(_NG - _KG, _NG):
        outs[g].wait()


def kernel(data, shape):
    planes = jnp.transpose(data, (0, 3, 1, 2)).reshape(_N, _D, _D)
    out = pl.pallas_call(
        _pad_kernel,
        in_specs=[pl.BlockSpec(memory_space=pl.ANY)],
        out_specs=pl.BlockSpec(memory_space=pl.ANY),
        out_shape=jax.ShapeDtypeStruct((_N, _M, _M), jnp.float32),
        scratch_shapes=[
            pltpu.VMEM((_K, _M, _M), jnp.float32),
            pltpu.SemaphoreType.DMA((_KG,)),
            pltpu.SemaphoreType.DMA((_KG,)),
        ],
    )(planes)
    return jnp.transpose(out.reshape(_B, _C, _M, _M), (0, 2, 3, 1))


# speedup vs baseline: 1.0589x; 1.0589x over previous
import jax
import jax.numpy as jnp
from jax.experimental import pallas as pl
from jax.experimental.pallas import tpu as pltpu

_B = 16
_C = 3
_D = 384
_M = 512
_P = _M - _D
_N = _B * _C
_SIZES = (2, 4, 8, 8, 8, 8, 6, 4)
_OFFS = tuple(sum(_SIZES[:i]) for i in range(len(_SIZES)))
_NG = len(_SIZES)
_KG = 3
_GMAX = max(_SIZES)
_K = _KG * _GMAX


def _fill_bank(buf, j):
    base = j * _GMAX
    buf[base : base + _GMAX, : _D, _D :] = jnp.full(
        (_GMAX, _D, _P), jnp.nan, jnp.float32
    )
    buf[base : base + _GMAX, _D :, :] = jnp.full(
        (_GMAX, _P, _M), jnp.nan, jnp.float32
    )


def _pad_kernel(d_hbm, o_hbm, buf, in_sems, out_sems):
    ins = [
        pltpu.make_async_copy(
            d_hbm.at[pl.ds(_OFFS[g], _SIZES[g])],
            buf.at[
                pl.ds((g % _KG) * _GMAX, _SIZES[g]), pl.ds(0, _D), pl.ds(0, _D)
            ],
            in_sems.at[g % _KG],
        )
        for g in range(_NG)
    ]
    outs = [
        pltpu.make_async_copy(
            buf.at[pl.ds((g % _KG) * _GMAX, _SIZES[g])],
            o_hbm.at[pl.ds(_OFFS[g], _SIZES[g])],
            out_sems.at[g % _KG],
        )
        for g in range(_NG)
    ]
    for g in range(_KG):
        ins[g].start(priority=g % 2)
    _fill_bank(buf, 0)
    for g in range(_NG):
        ins[g].wait()
        outs[g].start(priority=g % 2)
        if g + 1 < _KG:
            _fill_bank(buf, g + 1)
        gd = g - 1
        if gd >= 0 and gd + _KG < _NG:
            outs[gd].wait()
            ins[gd + _KG].start(priority=(gd + _KG) % 2)
    for g in range(_NG - _KG, _NG):
        outs[g].wait()


def kernel(data, shape):
    planes = jnp.transpose(data, (0, 3, 1, 2)).reshape(_N, _D, _D)
    out = pl.pallas_call(
        _pad_kernel,
        in_specs=[pl.BlockSpec(memory_space=pl.ANY)],
        out_specs=pl.BlockSpec(memory_space=pl.ANY),
        out_shape=jax.ShapeDtypeStruct((_N, _M, _M), jnp.float32),
        scratch_shapes=[
            pltpu.VMEM((_K, _M, _M), jnp.float32),
            pltpu.SemaphoreType.DMA((_KG,)),
            pltpu.SemaphoreType.DMA((_KG,)),
        ],
    )(planes)
    return jnp.transpose(out.reshape(_B, _C, _M, _M), (0, 2, 3, 1))
